# Initial kernel scaffold; baseline (speedup 1.0000x reference)
import jax
import jax.numpy as jnp
from jax.experimental import pallas as pl


def kernel(x_0, x_1, x_2, adjacency_0, adjacency_1, incidence_1, incidence_2, lin0_W, lin0_b, lin1_W, lin1_b, lin2_W, lin2_b, l0_W0, l0_b0, l0_W1, l0_b1, l1_W0, l1_b0, l1_W1, l1_b1, out0_W, out0_b, out1_W, out1_b, out2_W, out2_b):
    raise NotImplementedError("write your pallas kernel here")


# speedup vs baseline: 3.7242x; 3.7242x over previous
import functools

import jax
import jax.numpy as jnp
from jax.experimental import pallas as pl
from jax.experimental.pallas import tpu as pltpu

_BLK = 1024
_HASH_MULTS_A = (-1640531527, 999999937, 774856787, 162287629)
_HASH_MULTS_B = (1103515245, 69069, 1664525, 22695477)


def _rup(n, m):
    return ((n + m - 1) // m) * m


def _pad_for(n):
    if n >= _BLK:
        return _rup(n, _BLK), _BLK
    return _rup(n, 8), _rup(n, 8)


def _prep_body(x_ref, w1_ref, b1_ref, w2_ref, b2_ref, rh_ref, o_ref, *, n_real, blk):
    r = pl.program_id(0)
    rows = r * blk + jax.lax.broadcasted_iota(jnp.int32, (blk, 1), 0)
    ok = rows < n_real
    xv = jnp.where(ok, x_ref[...], 0.0)
    t = jnp.dot(xv, w1_ref[...], preferred_element_type=jnp.float32) + b1_ref[...]
    m = jnp.dot(t, w2_ref[...], preferred_element_type=jnp.float32) + b2_ref[...]
    m = jnp.where(ok, m, 0.0)
    o_ref[:, :128] = m.astype(jnp.bfloat16)
    o_ref[:, 128:] = rh_ref[...].astype(jnp.bfloat16)


def _edge1_a_body(a_ref, x_ref, za_ref, ap_ref, *, n_rows, n_cols, blk):
    j, k = pl.program_id(0), pl.program_id(1)
    rows = j * blk + jax.lax.broadcasted_iota(jnp.int32, (blk, 1), 0)
    cols = k * blk + jax.lax.broadcasted_iota(jnp.int32, (1, blk), 1)
    a = jnp.where((rows < n_rows) & (cols < n_cols), a_ref[...], 0.0)
    ab = a.astype(jnp.bfloat16)
    ap_ref[...] = ab

    @pl.when(k == 0)
    def _():
        za_ref[...] = jnp.zeros_like(za_ref)

    za_ref[...] += jnp.dot(ab, x_ref[...], preferred_element_type=jnp.float32)


def _edge1_i_body(i_ref, x_ref, zi_ref, ip_ref, *, n_rows, n_cols, blk, blkc):
    j, k = pl.program_id(0), pl.program_id(1)
    rows = k * blk + jax.lax.broadcasted_iota(jnp.int32, (blk, 1), 0)
    cols = j * blkc + jax.lax.broadcasted_iota(jnp.int32, (1, blkc), 1)
    a = jnp.where((rows < n_rows) & (cols < n_cols), i_ref[...], 0.0)
    ab = a.astype(jnp.bfloat16)
    ip_ref[...] = ab

    @pl.when(k == 0)
    def _():
        zi_ref[...] = jnp.zeros_like(zi_ref)

    zi_ref[...] += jax.lax.dot_general(
        ab, x_ref[...], (((0,), (0,)), ((), ())),
        preferred_element_type=jnp.float32)


def _edge_a_body(ap_ref, x_ref, za_ref):
    k = pl.program_id(1)

    @pl.when(k == 0)
    def _():
        za_ref[...] = jnp.zeros_like(za_ref)

    za_ref[...] += jnp.dot(ap_ref[...], x_ref[...],
                           preferred_element_type=jnp.float32)


def _edge_i_body(ip_ref, x_ref, zi_ref):
    k = pl.program_id(1)

    @pl.when(k == 0)
    def _():
        zi_ref[...] = jnp.zeros_like(zi_ref)

    zi_ref[...] += jax.lax.dot_general(
        ip_ref[...], x_ref[...], (((0,), (0,)), ((), ())),
        preferred_element_type=jnp.float32)


def _node_a_body(ap_ref, y_ref, o_ref):
    k = pl.program_id(1)

    @pl.when(k == 0)
    def _():
        o_ref[...] = jnp.zeros_like(o_ref)

    o_ref[...] += jax.lax.dot_general(
        ap_ref[...], y_ref[...], (((0,), (0,)), ((), ())),
        preferred_element_type=jnp.float32)


def _node_i_body(ip_ref, y_ref, p_ref, o_ref):
    k = pl.program_id(1)

    @pl.when(k == 0)
    def _():
        o_ref[...] = p_ref[...]

    o_ref[...] += jnp.dot(ip_ref[...], y_ref[...],
                          preferred_element_type=jnp.float32)


def _mix(cols, mults):
    acc = cols[0] * mults[0]
    for c, m in zip(cols[1:], mults[1:]):
        acc = acc + c * m
    return acc


def _dedup_body(hc_ref, hr_ref, v_ref, *, jb, n0, n0p, n1):
    j = pl.program_id(0)

    def jrow(t):
        return hr_ref[pl.ds(t, 1), pl.ds(j * jb, jb)].astype(jnp.int32)

    s_j = jrow(7)
    cj1 = _mix([s_j, jrow(0), jrow(1), jrow(2)], _HASH_MULTS_A)
    cj2 = _mix([jrow(3), jrow(4), jrow(5), jrow(6)], _HASH_MULTS_B)
    jj = j * jb + jax.lax.broadcasted_iota(jnp.int32, (1, jb), 1)

    def body(c, acc):
        hi = hc_ref[pl.ds(c * jb, jb), :].astype(jnp.int32)

        def col(t):
            return jax.lax.slice(hi, (0, t), (jb, t + 1))

        ci1 = _mix([col(7), col(0), col(1), col(2)], _HASH_MULTS_A)
        ci2 = _mix([col(3), col(4), col(5), col(6)], _HASH_MULTS_B)
        ii = c * jb + jax.lax.broadcasted_iota(jnp.int32, (jb, 1), 0)
        hit = (ci1 == cj1) & (ci2 == cj2) & (ii < jj)
        return acc | jnp.any(hit, axis=0, keepdims=True).astype(jnp.int32)

    dup = jax.lax.fori_loop(0, j + 1, body, jnp.zeros((1, jb), jnp.int32))
    real = (jj < n0) | ((jj >= n0p) & (jj < n0p + n1))
    valid = (s_j > 0) & (dup == 0) & real
    v_ref[...] = valid.astype(jnp.float32)


def _mid_edge_body(zz_ref, s_ref, v_ref, w_ref, b_ref, y_ref, *, with_valid_col,
                   blk):
    valid = v_ref[...]
    d_e = jnp.maximum(s_ref[...] * valid, 1.0)
    z1 = jnp.maximum(jnp.where(valid > 0, zz_ref[:, :128], 0.0) / d_e, 0.0)
    t = jnp.dot(z1, w_ref[...], preferred_element_type=jnp.float32) + b_ref[...]
    t = t * valid
    y_ref[:, :128] = t.astype(jnp.bfloat16)
    if with_valid_col:
        lane = jax.lax.broadcasted_iota(jnp.int32, (blk, 128), 1)
        y_ref[:, 128:] = jnp.where(lane == 0, valid, 0.0).astype(jnp.bfloat16)


def _mid_node_body(h_ref, w_ref, b_ref, x_ref, *, n_real, blk):
    r = pl.program_id(0)
    rows = r * blk + jax.lax.broadcasted_iota(jnp.int32, (blk, 1), 0)
    ok = rows < n_real
    dv = jnp.maximum(h_ref[:, 128:129], 1.0)
    h = jnp.maximum(jnp.where(ok, h_ref[:, :128], 0.0) / dv, 0.0)
    m = jnp.dot(h, w_ref[...], preferred_element_type=jnp.float32) + b_ref[...]
    x_ref[...] = jnp.where(ok, m, 0.0).astype(jnp.bfloat16)


def _final_body(h_ref, dv_ref, x1_ref, x2_ref,
                w0_ref, b0_ref,
                l1w_ref, l1b_ref, o1w_ref, o1b_ref,
                l2w_ref, l2b_ref, o2w_ref, o2b_ref,
                out_ref, *, n0, n1, n2):
    rows = jax.lax.broadcasted_iota(jnp.int32, (h_ref.shape[0], 1), 0)
    ok = rows < n0
    dv = jnp.maximum(dv_ref[...], 1.0)
    h = jnp.maximum(jnp.where(ok, h_ref[...], 0.0) / dv, 0.0)
    mh = jnp.sum(h, axis=0, keepdims=True) / n0
    o0 = jnp.dot(mh, w0_ref[...], preferred_element_type=jnp.float32) + b0_ref[...]

    mx1 = jnp.sum(x1_ref[...], axis=0, keepdims=True) / n1
    h1 = jnp.dot(mx1, l1w_ref[...], preferred_element_type=jnp.float32) + l1b_ref[...]
    o1 = jnp.dot(h1, o1w_ref[...], preferred_element_type=jnp.float32) + o1b_ref[...]

    mx2 = jnp.sum(x2_ref[...], axis=0, keepdims=True) / n2
    h2 = jnp.dot(mx2, l2w_ref[...], preferred_element_type=jnp.float32) + l2b_ref[...]
    o2 = jnp.dot(h2, o2w_ref[...], preferred_element_type=jnp.float32) + o2b_ref[...]

    out_ref[...] = o0 + o1 + o2


def _mm_params():
    return pltpu.CompilerParams(
        dimension_semantics=("parallel", "arbitrary"))


def kernel(x_0, x_1, x_2, adjacency_0, adjacency_1, incidence_1, incidence_2,
           lin0_W, lin0_b, lin1_W, lin1_b, lin2_W, lin2_b,
           l0_W0, l0_b0, l0_W1, l0_b1, l1_W0, l1_b0, l1_W1, l1_b1,
           out0_W, out0_b, out1_W, out1_b, out2_W, out2_b):
    f32, bf16 = jnp.float32, jnp.bfloat16
    n0, n1, n2 = x_0.shape[0], x_1.shape[0], x_2.shape[0]
    n0p, blk = _pad_for(n0)
    n1p = _rup(n1, blk) if n1 >= blk else _rup(n1, 8)
    blk1 = blk if n1p % blk == 0 else n1p
    mp = n0p + n1p
    jb = 512 if mp % 512 == 0 else mp
    mblk = 1024 if mp % 1024 == 0 else mp

    g0, g1 = n0p // blk, n1p // blk1

    key = jax.random.key(20260805)
    rvals = jax.random.randint(key, (n0, 7), 0, 256).astype(f32)
    rh = jnp.zeros((n0p, 128), f32)
    rh = rh.at[:n0, :7].set(rvals)
    rh = rh.at[:n0, 7].set(1.0)

    b_ = lambda v: v.reshape(1, -1)

    x1m = pl.pallas_call(
        functools.partial(_prep_body, n_real=n0, blk=blk),
        grid=(g0,),
        in_specs=[
            pl.BlockSpec((blk, 128), lambda r: (r, 0)),
            pl.BlockSpec((128, 128), lambda r: (0, 0)),
            pl.BlockSpec((1, 128), lambda r: (0, 0)),
            pl.BlockSpec((128, 128), lambda r: (0, 0)),
            pl.BlockSpec((1, 128), lambda r: (0, 0)),
            pl.BlockSpec((blk, 128), lambda r: (r, 0)),
        ],
        out_specs=pl.BlockSpec((blk, 256), lambda r: (r, 0)),
        out_shape=jax.ShapeDtypeStruct((n0p, 256), bf16),
    )(x_0, lin0_W, b_(lin0_b), l0_W0, b_(l0_b0), rh)

    za, ap = pl.pallas_call(
        functools.partial(_edge1_a_body, n_rows=n0, n_cols=n0, blk=blk),
        grid=(g0, g0),
        in_specs=[
            pl.BlockSpec((blk, blk), lambda j, k: (j, k)),
            pl.BlockSpec((blk, 256), lambda j, k: (k, 0)),
        ],
        out_specs=[
            pl.BlockSpec((blk, 256), lambda j, k: (j, 0)),
            pl.BlockSpec((blk, blk), lambda j, k: (j, k)),
        ],
        out_shape=[
            jax.ShapeDtypeStruct((n0p, 256), f32),
            jax.ShapeDtypeStruct((n0p, n0p), bf16),
        ],
        compiler_params=_mm_params(),
    )(adjacency_0, x1m)

    zi, ip = pl.pallas_call(
        functools.partial(_edge1_i_body, n_rows=n0, n_cols=n1, blk=blk,
                          blkc=blk1),
        grid=(g1, g0),
        in_specs=[
            pl.BlockSpec((blk, blk1), lambda j, k: (k, j)),
            pl.BlockSpec((blk, 256), lambda j, k: (k, 0)),
        ],
        out_specs=[
            pl.BlockSpec((blk1, 256), lambda j, k: (j, 0)),
            pl.BlockSpec((blk, blk1), lambda j, k: (k, j)),
        ],
        out_shape=[
            jax.ShapeDtypeStruct((n1p, 256), f32),
            jax.ShapeDtypeStruct((n0p, n1p), bf16),
        ],
        compiler_params=_mm_params(),
    )(incidence_1, x1m)

    zz = jnp.concatenate([za, zi], axis=0)
    hs = zz[:, 128:136]
    hst = hs.T

    valid_row = pl.pallas_call(
        functools.partial(_dedup_body, jb=jb, n0=n0, n0p=n0p, n1=n1),
        grid=(mp // jb,),
        in_specs=[
            pl.BlockSpec((mp, 8), lambda j: (0, 0)),
            pl.BlockSpec((8, mp), lambda j: (0, 0)),
        ],
        out_specs=pl.BlockSpec((1, jb), lambda j: (0, j)),
        out_shape=jax.ShapeDtypeStruct((1, mp), f32),
    )(hs, hst)
    valid = valid_row.reshape(mp, 1)

    def mid_edge(zzv, scol, w, b, with_valid_col):
        return pl.pallas_call(
            functools.partial(_mid_edge_body, with_valid_col=with_valid_col,
                              blk=mblk),
            grid=(mp // mblk,),
            in_specs=[
                pl.BlockSpec((mblk, zzv.shape[1]), lambda r: (r, 0)),
                pl.BlockSpec((mblk, 1), lambda r: (r, 0)),
                pl.BlockSpec((mblk, 1), lambda r: (r, 0)),
                pl.BlockSpec((128, 128), lambda r: (0, 0)),
                pl.BlockSpec((1, 128), lambda r: (0, 0)),
            ],
            out_specs=pl.BlockSpec((mblk, 256 if with_valid_col else 128),
                                   lambda r: (r, 0)),
            out_shape=jax.ShapeDtypeStruct(
                (mp, 256 if with_valid_col else 128), bf16),
        )(zzv, scol, valid, w, b_(b))

    def node_pass(ya, yi, width):
        part = pl.pallas_call(
            _node_a_body,
            grid=(g0, g0),
            in_specs=[
                pl.BlockSpec((blk, blk), lambda i, k: (k, i)),
                pl.BlockSpec((blk, width), lambda i, k: (k, 0)),
            ],
            out_specs=pl.BlockSpec((blk, width), lambda i, k: (i, 0)),
            out_shape=jax.ShapeDtypeStruct((n0p, width), f32),
            compiler_params=_mm_params(),
        )(ap, ya)
        return pl.pallas_call(
            _node_i_body,
            grid=(g0, g1),
            in_specs=[
                pl.BlockSpec((blk, blk1), lambda i, k: (i, k)),
                pl.BlockSpec((blk1, width), lambda i, k: (k, 0)),
                pl.BlockSpec((blk, width), lambda i, k: (i, 0)),
            ],
            out_specs=pl.BlockSpec((blk, width), lambda i, k: (i, 0)),
            out_shape=jax.ShapeDtypeStruct((n0p, width), f32),
            compiler_params=_mm_params(),
        )(ip, yi, part)

    def edge_pass(xm):
        zae = pl.pallas_call(
            _edge_a_body,
            grid=(g0, g0),
            in_specs=[
                pl.BlockSpec((blk, blk), lambda j, k: (j, k)),
                pl.BlockSpec((blk, 128), lambda j, k: (k, 0)),
            ],
            out_specs=pl.BlockSpec((blk, 128), lambda j, k: (j, 0)),
            out_shape=jax.ShapeDtypeStruct((n0p, 128), f32),
            compiler_params=_mm_params(),
        )(ap, xm)
        zie = pl.pallas_call(
            _edge_i_body,
            grid=(g1, g0),
            in_specs=[
                pl.BlockSpec((blk, blk1), lambda j, k: (k, j)),
                pl.BlockSpec((blk, 128), lambda j, k: (k, 0)),
            ],
            out_specs=pl.BlockSpec((blk1, 128), lambda j, k: (j, 0)),
            out_shape=jax.ShapeDtypeStruct((n1p, 128), f32),
            compiler_params=_mm_params(),
        )(ip, xm)
        return jnp.concatenate([zae, zie], axis=0)

    scol = zz[:, 135:136]

    y1 = mid_edge(zz, scol, l0_W1, l0_b1, with_valid_col=True)
    h1n = node_pass(y1[:n0p], y1[n0p:], 256)

    x2m = pl.pallas_call(
        functools.partial(_mid_node_body, n_real=n0, blk=blk),
        grid=(g0,),
        in_specs=[
            pl.BlockSpec((blk, 256), lambda r: (r, 0)),
            pl.BlockSpec((128, 128), lambda r: (0, 0)),
            pl.BlockSpec((1, 128), lambda r: (0, 0)),
        ],
        out_specs=pl.BlockSpec((blk, 128), lambda r: (r, 0)),
        out_shape=jax.ShapeDtypeStruct((n0p, 128), bf16),
    )(h1n, l1_W0, b_(l1_b0))

    zz2 = edge_pass(x2m)
    y2 = mid_edge(zz2, scol, l1_W1, l1_b1, with_valid_col=False)
    h2n = node_pass(y2[:n0p], y2[n0p:], 128)

    out = pl.pallas_call(
        functools.partial(_final_body, n0=float(n0), n1=float(n1), n2=float(n2)),
        out_shape=jax.ShapeDtypeStruct((1, 1), f32),
    )(h2n, h1n[:, 128:129], x_1, x_2,
      out0_W, b_(out0_b), lin1_W, b_(lin1_b), out1_W, b_(out1_b),
      lin2_W, b_(lin2_b), out2_W, b_(out2_b))

    return out.reshape((1,))
